# Initial kernel scaffold; baseline (speedup 1.0000x reference)
import jax
import jax.numpy as jnp
from jax.experimental import pallas as pl


def kernel(x, edge_index, id, W, W_id, att):
    raise NotImplementedError("write your pallas kernel here")


# speedup vs baseline: 13.6424x; 13.6424x over previous
import functools

import jax
import jax.numpy as jnp
from jax import lax
from jax.experimental import pallas as pl
from jax.experimental.pallas import tpu as pltpu
from jax.experimental.pallas import tpu_sc as plsc

N = 10000
E = 320000
D = 128
NUM_ID = 1000
NEG_SLOPE = 0.2

NC = 2
NS = 16
NW = NC * NS
CH = 128
NCHUNK = 80
EPT = NCHUNK * CH
E_PAD = NW * EPT
N_PAD = 10240
DROW = N_PAD // D
ROWS_PER_TILE = N_PAD // NS
RP = 128


def _dense_body(id_ref, x_ref, w_ref, wid_ref, att2_ref,
                h_out_ref, b_ref, c_ref, es_ref, dx_ref):
    dx_ref[...] = jnp.zeros_like(dx_ref)

    def add_row(j, carry):
        r = id_ref[j]
        dx_ref[pl.ds(r, 1), :] = dx_ref[pl.ds(r, 1), :] + x_ref[pl.ds(r, 1), :]
        return carry

    lax.fori_loop(0, NUM_ID, add_row, 0)

    h = jnp.dot(x_ref[...], w_ref[...], preferred_element_type=jnp.float32)
    h = h + jnp.dot(dx_ref[...], wid_ref[...],
                    preferred_element_type=jnp.float32)

    cb = jnp.dot(h, att2_ref[...], preferred_element_type=jnp.float32)
    c = cb[:, 0:1]
    b = cb[:, 1:2]
    a_self = c + b
    a_self = jnp.where(a_self >= 0.0, a_self, NEG_SLOPE * a_self)
    es = jnp.exp(a_self)

    h_out_ref[...] = h
    b_ref[...] = b
    c_ref[...] = c
    es_ref[...] = es


def _dense(x, node_id, W, W_id, att2):
    return pl.pallas_call(
        _dense_body,
        out_shape=(
            jax.ShapeDtypeStruct((N, D), jnp.float32),
            jax.ShapeDtypeStruct((N, 1), jnp.float32),
            jax.ShapeDtypeStruct((N, 1), jnp.float32),
            jax.ShapeDtypeStruct((N, 1), jnp.float32),
        ),
        in_specs=[
            pl.BlockSpec(memory_space=pltpu.SMEM),
            pl.BlockSpec(memory_space=pltpu.VMEM),
            pl.BlockSpec(memory_space=pltpu.VMEM),
            pl.BlockSpec(memory_space=pltpu.VMEM),
            pl.BlockSpec(memory_space=pltpu.VMEM),
        ],
        scratch_shapes=[pltpu.VMEM((N, D), jnp.float32)],
    )(node_id, x, W, W_id, att2)


def _edge_body(src_hbm, dst_hbm, b_hbm, c_hbm, h_hbm,
               out_hbm, den_hbm,
               src_v, dst_v, b_v, c_v, rows_v, den_v, iota_v, acc_sh,
               den_sh, sem):
    cid = lax.axis_index("c")
    sid = lax.axis_index("s")
    wid = cid * NS + sid

    zv = jnp.zeros((16,), jnp.float32)

    def zrow(i, carry):
        for d in range(D // 16):
            rows_v[i, pl.ds(d * 16, 16)] = zv
        return carry

    lax.fori_loop(0, CH, zrow, 0)

    def zden(i, carry):
        for d in range(D // 16):
            den_v[i, pl.ds(d * 16, 16)] = zv
        return carry

    lax.fori_loop(0, DROW, zden, 0)

    for t in range(ROWS_PER_TILE // RP):
        pltpu.sync_copy(rows_v,
                        acc_sh.at[pl.ds(sid * ROWS_PER_TILE + t * RP, RP)])

    @pl.when(sid == 0)
    def _zero_den():
        pltpu.sync_copy(rows_v.at[pl.ds(0, DROW)], den_sh)

    for t in range(DROW // 16):
        iota_v[0, pl.ds(t * 16, 16)] = lax.iota(jnp.int32, 16) + (t * 16)

    pltpu.sync_copy(b_hbm, b_v)
    pltpu.sync_copy(c_hbm, c_v)
    plsc.subcore_barrier()

    def chunk(j, carry):
        pltpu.sync_copy(src_hbm.at[wid, j], src_v.at[0])
        pltpu.sync_copy(dst_hbm.at[wid, j], dst_v.at[0])
        pltpu.async_copy(h_hbm.at[src_v.at[0]], rows_v, sem).wait()

        for v in range(CH // 16):
            si = src_v[0, pl.ds(v * 16, 16)]
            di = dst_v[0, pl.ds(v * 16, 16)]
            bs = plsc.load_gather(b_v, [si])
            cs = plsc.load_gather(c_v, [di])
            al = bs + cs
            al = jnp.where(al >= 0.0, al, NEG_SLOPE * al)
            ex = jnp.where(si != di, jnp.exp(al), 0.0)
            plsc.addupdate_scatter(
                den_v, [lax.shift_right_logical(di, 7),
                        lax.bitwise_and(di, 127)], ex)
            for l in range(16):
                k = v * 16 + l
                e = ex[l]
                for d in range(D // 16):
                    rows_v[k, pl.ds(d * 16, 16)] = (
                        rows_v[k, pl.ds(d * 16, 16)] * e)

        pltpu.sync_copy(rows_v, acc_sh.at[dst_v.at[0]], add=True)
        return carry

    lax.fori_loop(0, NCHUNK, chunk, 0)

    pltpu.sync_copy(den_v, den_sh.at[iota_v.at[0]], add=True)
    plsc.subcore_barrier()

    for t in range(ROWS_PER_TILE // RP):
        r0 = sid * ROWS_PER_TILE + t * RP
        pltpu.sync_copy(acc_sh.at[pl.ds(r0, RP)], rows_v)
        pltpu.sync_copy(rows_v, out_hbm.at[cid, pl.ds(r0, RP)])

    @pl.when(sid == 0)
    def _write_den():
        pltpu.sync_copy(den_sh, den_v)
        pltpu.sync_copy(den_v, den_hbm.at[cid])


def _edges(srcp, dstp, b, c, h):
    mesh = plsc.VectorSubcoreMesh(core_axis_name="c", subcore_axis_name="s")
    f = functools.partial(
        pl.kernel,
        mesh=mesh,
        compiler_params=pltpu.CompilerParams(needs_layout_passes=False),
        out_type=(
            jax.ShapeDtypeStruct((NC, N_PAD, D), jnp.float32),
            jax.ShapeDtypeStruct((NC, DROW, D), jnp.float32),
        ),
        scratch_types=[
            pltpu.VMEM((1, CH), jnp.int32),
            pltpu.VMEM((1, CH), jnp.int32),
            pltpu.VMEM((N,), jnp.float32),
            pltpu.VMEM((N,), jnp.float32),
            pltpu.VMEM((CH, D), jnp.float32),
            pltpu.VMEM((DROW, D), jnp.float32),
            pltpu.VMEM((1, DROW), jnp.int32),
            pltpu.VMEM_SHARED((N_PAD, D), jnp.float32),
            pltpu.VMEM_SHARED((DROW, D), jnp.float32),
            pltpu.SemaphoreType.DMA,
        ],
    )(_edge_body)
    return f(srcp, dstp, b, c, h)


def _combine_body(acc_ref, den_ref, h_ref, es_ref, out_ref):
    h = h_ref[...]
    es = es_ref[...]
    num = acc_ref[0, :N, :] + acc_ref[1, :N, :] + es * h
    den = den_ref[0] + den_ref[1] + es + 1e-16
    out_ref[...] = num / den


def _combine(acc, den, h, es):
    return pl.pallas_call(
        _combine_body,
        out_shape=jax.ShapeDtypeStruct((N, D), jnp.float32),
    )(acc, den, h, es)


def kernel(x, edge_index, id, W, W_id, att):
    att2 = jnp.transpose(att.reshape(2, D))
    h, b, c, es = _dense(x, id, W, W_id, att2)

    src = edge_index[0]
    dst = edge_index[1]
    pad = jnp.zeros((E_PAD - E,), jnp.int32)
    srcp = jnp.concatenate([src, pad]).reshape(NW, NCHUNK, CH)
    dstp = jnp.concatenate([dst, pad]).reshape(NW, NCHUNK, CH)

    acc, den = _edges(srcp, dstp, b.reshape(N), c.reshape(N), h)
    den_n = den.reshape(NC, N_PAD, 1)[:, :N, :]
    return _combine(acc, den_n, h, es)
